# baseline (speedup 1.0000x reference)
import jax
import jax.numpy as jnp
from jax import lax
from jax.experimental import pallas as pl
from jax.experimental.pallas import tpu as pltpu
from jax.experimental.pallas import tpu_sc as plsc

_N_IDS = 8192
_N_BINS = 256
_L = 16
_D = 64
_H = 64
_O = 8


def _fused_body(ids_hbm, emb_hbm, w1_hbm, b1_hbm, w2_hbm, b2_hbm, out_hbm,
                ids_v, cnt_v, emb_v, w1_v, w2_v, b1_v, b2_v, v_v,
                sem_ids, sem_emb, sem_w):
    s = lax.axis_index("s")

    @pl.when(s == 0)
    def _():
        zeros16 = jnp.zeros((_L,), jnp.float32)
        cp_ids = pltpu.async_copy(ids_hbm, ids_v, sem_ids)
        cp_emb = pltpu.async_copy(emb_hbm, emb_v, sem_emb)
        b2_v[pl.ds(0, _L)] = zeros16
        for j in range(_O * _H, _L * _H, _L):
            w2_v[pl.ds(j, _L)] = zeros16
        pltpu.async_copy(w1_hbm, w1_v, sem_w)
        pltpu.async_copy(w2_hbm, w2_v.at[pl.ds(0, _O * _H)], sem_w)
        pltpu.async_copy(b1_hbm, b1_v, sem_w)
        pltpu.async_copy(b2_hbm, b2_v.at[pl.ds(0, _O)], sem_w)

        cp_ids.wait()
        for j in range(_N_BINS // _L):
            cnt_v[pl.ds(j * _L, _L)] = zeros16
        ones = jnp.ones((_L,), jnp.float32)
        for j in range(_N_IDS // _L):
            plsc.addupdate_scatter(cnt_v, [ids_v[pl.ds(j * _L, _L)]], ones)

        cp_emb.wait()
        acc = [zeros16 for _ in range(_D // _L)]
        for jb in range(_N_BINS // _L):
            cnt16 = cnt_v[pl.ds(jb * _L, _L)] * (1.0 / _N_IDS)
            for j in range(_L):
                c = cnt16[j]
                base = (jb * _L + j) * _D
                for cc in range(_D // _L):
                    acc[cc] = acc[cc] + c * emb_v[pl.ds(base + cc * _L, _L)]

        pltpu.make_async_copy(w1_hbm, w1_v, sem_w).wait()
        pltpu.make_async_copy(w2_hbm, w2_v.at[pl.ds(0, _O * _H)], sem_w).wait()
        pltpu.make_async_copy(b1_hbm, b1_v, sem_w).wait()
        pltpu.make_async_copy(b2_hbm, b2_v.at[pl.ds(0, _O)], sem_w).wait()

        lane = lax.broadcasted_iota(jnp.int32, (_L,), 0)
        hs = []
        for jb in range(_H // _L):
            hj = b1_v[pl.ds(jb * _L, _L)]
            col = lane * _H + jb * _L * _H
            for k in range(_D):
                ek = acc[k // _L][k % _L]
                hj = hj + ek * plsc.load_gather(w1_v, [col + k])
            hs.append(jnp.maximum(hj, 0.0))

        v = b2_v[pl.ds(0, _L)]
        colw2 = lane * _H
        for k in range(_H):
            hk = hs[k // _L][k % _L]
            v = v + hk * plsc.load_gather(w2_v, [colw2 + k])
        v_v[pl.ds(0, _L)] = v
        pltpu.sync_copy(v_v.at[pl.ds(0, _O)], out_hbm)


_fused = pl.kernel(
    _fused_body,
    mesh=plsc.VectorSubcoreMesh(core_axis_name="c", subcore_axis_name="s",
                                num_cores=1),
    out_type=jax.ShapeDtypeStruct((_O,), jnp.float32),
    scratch_types=[
        pltpu.VMEM((_N_IDS,), jnp.int32),
        pltpu.VMEM((_N_BINS,), jnp.float32),
        pltpu.VMEM((_N_BINS * _D,), jnp.float32),
        pltpu.VMEM((_H * _D,), jnp.float32),
        pltpu.VMEM((_L * _H,), jnp.float32),
        pltpu.VMEM((_H,), jnp.float32),
        pltpu.VMEM((_L,), jnp.float32),
        pltpu.VMEM((_L,), jnp.float32),
        pltpu.SemaphoreType.DMA,
        pltpu.SemaphoreType.DMA,
        pltpu.SemaphoreType.DMA,
    ],
    compiler_params=pltpu.CompilerParams(needs_layout_passes=False,
                                         use_tc_tiling_on_sc=False),
)


def kernel(ids, emb, W1, b1, W2, b2):
    ids32 = ids.astype(jnp.int32)
    return _fused(ids32, emb.reshape(-1), W1.reshape(-1), b1,
                  W2.reshape(-1), b2)


# speedup vs baseline: 1.4609x; 1.4609x over previous
import jax
import jax.numpy as jnp
from jax import lax
from jax.experimental import pallas as pl
from jax.experimental.pallas import tpu as pltpu
from jax.experimental.pallas import tpu_sc as plsc

_N_IDS = 8192
_N_BINS = 256
_N_SUB = 16
_IDS_PER_SUB = _N_IDS // _N_SUB
_BINS_PER_SUB = _N_BINS // _N_SUB
_L = 16
_D = 64
_H = 64
_O = 8
_HSPLIT = 4


def _fused_body(ids_hbm, emb_hbm, w1_hbm, b1_hbm, w2_hbm, b2_hbm, out_hbm,
                ids_v, cnt_v, cnt16x16_v, emb_v, eloc_v, e16_v,
                w1_v, w2_v, b1_v, b2_v, v_v,
                cnt_slots, e_slots, sem_ids, sem_emb, sem_w):
    s = lax.axis_index("s")
    zeros16 = jnp.zeros((_L,), jnp.float32)

    cp_ids = pltpu.async_copy(
        ids_hbm.at[pl.ds(s * _IDS_PER_SUB, _IDS_PER_SUB)], ids_v, sem_ids)
    cp_emb = pltpu.async_copy(
        emb_hbm.at[pl.ds(s * _BINS_PER_SUB * _D, _BINS_PER_SUB * _D)],
        emb_v, sem_emb)

    @pl.when(s == 0)
    def _():
        b2_v[pl.ds(0, _L)] = zeros16
        for j in range(_O * _H, _L * _H, _L):
            w2_v[pl.ds(j, _L)] = zeros16
        pltpu.async_copy(w1_hbm, w1_v, sem_w)
        pltpu.async_copy(w2_hbm, w2_v.at[pl.ds(0, _O * _H)], sem_w)
        pltpu.async_copy(b1_hbm, b1_v, sem_w)
        pltpu.async_copy(b2_hbm, b2_v.at[pl.ds(0, _O)], sem_w)

    for j in range(_HSPLIT * _N_BINS // _L):
        cnt_v[pl.ds(j * _L, _L)] = zeros16
    cp_ids.wait()
    ones = jnp.ones((_L,), jnp.float32)
    nvec = _IDS_PER_SUB // _L
    for j in range(nvec):
        off = jnp.int32((j % _HSPLIT) * _N_BINS)
        plsc.addupdate_scatter(
            cnt_v, [ids_v[pl.ds(j * _L, _L)] + off], ones)
    for j in range(_N_BINS // _L):
        tot = cnt_v[pl.ds(j * _L, _L)]
        for b in range(1, _HSPLIT):
            tot = tot + cnt_v[pl.ds(b * _N_BINS + j * _L, _L)]
        cnt_v[pl.ds(j * _L, _L)] = tot
    pltpu.sync_copy(cnt_v.at[pl.ds(0, _N_BINS)], cnt_slots.at[s])
    plsc.subcore_barrier()

    pltpu.sync_copy(cnt_slots.at[:, pl.ds(s * _BINS_PER_SUB, _BINS_PER_SUB)],
                    cnt16x16_v)
    c_a = cnt16x16_v[0, pl.ds(0, _L)]
    c_b = cnt16x16_v[1, pl.ds(0, _L)]
    for r in range(2, _N_SUB, 2):
        c_a = c_a + cnt16x16_v[r, pl.ds(0, _L)]
        c_b = c_b + cnt16x16_v[r + 1, pl.ds(0, _L)]
    cnt16 = (c_a + c_b) * (1.0 / _N_IDS)

    cp_emb.wait()
    acc = [zeros16 for _ in range(_D // _L)]
    for j in range(_BINS_PER_SUB):
        c = cnt16[j]
        for cc in range(_D // _L):
            acc[cc] = acc[cc] + c * emb_v[pl.ds(j * _D + cc * _L, _L)]
    for cc in range(_D // _L):
        eloc_v[pl.ds(cc * _L, _L)] = acc[cc]
    pltpu.sync_copy(eloc_v, e_slots.at[s])
    plsc.subcore_barrier()

    @pl.when(s == 0)
    def _():
        pltpu.sync_copy(e_slots, e16_v)
        e_blk = []
        for cc in range(_D // _L):
            ea = e16_v[0, pl.ds(cc * _L, _L)]
            eb = e16_v[1, pl.ds(cc * _L, _L)]
            for r in range(2, _N_SUB, 2):
                ea = ea + e16_v[r, pl.ds(cc * _L, _L)]
                eb = eb + e16_v[r + 1, pl.ds(cc * _L, _L)]
            e_blk.append(ea + eb)

        pltpu.make_async_copy(w1_hbm, w1_v, sem_w).wait()
        pltpu.make_async_copy(w2_hbm, w2_v.at[pl.ds(0, _O * _H)], sem_w).wait()
        pltpu.make_async_copy(b1_hbm, b1_v, sem_w).wait()
        pltpu.make_async_copy(b2_hbm, b2_v.at[pl.ds(0, _O)], sem_w).wait()

        lane = lax.broadcasted_iota(jnp.int32, (_L,), 0)
        hs = []
        for jb in range(_H // _L):
            col = lane * _H + jb * _L * _H
            parts = [None] * 4
            for k in range(_D):
                ek = e_blk[k // _L][k % _L]
                t = ek * plsc.load_gather(w1_v, [col + k])
                p = k % 4
                parts[p] = t if parts[p] is None else parts[p] + t
            hj = (b1_v[pl.ds(jb * _L, _L)]
                  + ((parts[0] + parts[1]) + (parts[2] + parts[3])))
            hs.append(jnp.maximum(hj, 0.0))

        colw2 = lane * _H
        parts = [None] * 4
        for k in range(_H):
            hk = hs[k // _L][k % _L]
            t = hk * plsc.load_gather(w2_v, [colw2 + k])
            p = k % 4
            parts[p] = t if parts[p] is None else parts[p] + t
        v = (b2_v[pl.ds(0, _L)]
             + ((parts[0] + parts[1]) + (parts[2] + parts[3])))
        v_v[pl.ds(0, _L)] = v
        pltpu.sync_copy(v_v.at[pl.ds(0, _O)], out_hbm)


_fused = pl.kernel(
    _fused_body,
    mesh=plsc.VectorSubcoreMesh(core_axis_name="c", subcore_axis_name="s",
                                num_cores=1),
    out_type=jax.ShapeDtypeStruct((_O,), jnp.float32),
    scratch_types=[
        pltpu.VMEM((_IDS_PER_SUB,), jnp.int32),
        pltpu.VMEM((_HSPLIT * _N_BINS,), jnp.float32),
        pltpu.VMEM((_N_SUB, _BINS_PER_SUB), jnp.float32),
        pltpu.VMEM((_BINS_PER_SUB * _D,), jnp.float32),
        pltpu.VMEM((_D,), jnp.float32),
        pltpu.VMEM((_N_SUB, _D), jnp.float32),
        pltpu.VMEM((_H * _D,), jnp.float32),
        pltpu.VMEM((_L * _H,), jnp.float32),
        pltpu.VMEM((_H,), jnp.float32),
        pltpu.VMEM((_L,), jnp.float32),
        pltpu.VMEM((_L,), jnp.float32),
        pltpu.VMEM_SHARED((_N_SUB, _N_BINS), jnp.float32),
        pltpu.VMEM_SHARED((_N_SUB, _D), jnp.float32),
        pltpu.SemaphoreType.DMA,
        pltpu.SemaphoreType.DMA,
        pltpu.SemaphoreType.DMA,
    ],
    compiler_params=pltpu.CompilerParams(needs_layout_passes=False,
                                         use_tc_tiling_on_sc=False,
                                         disable_bounds_checks=True),
)


def kernel(ids, emb, W1, b1, W2, b2):
    ids32 = ids.astype(jnp.int32)
    return _fused(ids32, emb.reshape(-1), W1.reshape(-1), b1,
                  W2.reshape(-1), b2)


# speedup vs baseline: 1.5722x; 1.0762x over previous
import jax
import jax.numpy as jnp
from jax import lax
from jax.experimental import pallas as pl
from jax.experimental.pallas import tpu as pltpu
from jax.experimental.pallas import tpu_sc as plsc

_N_IDS = 8192
_N_BINS = 256
_N_WORKERS = 32
_IDS_PER_WORKER = _N_IDS // _N_WORKERS
_L = 16


def _hist_body(ids_hbm, out_hbm, ids_v, counts_v):
    wid = lax.axis_index("s") * 2 + lax.axis_index("c")
    pltpu.sync_copy(ids_hbm.at[pl.ds(wid * _IDS_PER_WORKER, _IDS_PER_WORKER)],
                    ids_v)
    zeros = jnp.zeros((_L,), jnp.float32)
    for j in range(_N_BINS // _L):
        counts_v[pl.ds(j * _L, _L)] = zeros
    ones = jnp.ones((_L,), jnp.float32)
    for j in range(_IDS_PER_WORKER // _L):
        plsc.addupdate_scatter(counts_v, [ids_v[pl.ds(j * _L, _L)]], ones)
    pltpu.sync_copy(counts_v, out_hbm.at[wid])


_hist = pl.kernel(
    _hist_body,
    mesh=plsc.VectorSubcoreMesh(core_axis_name="c", subcore_axis_name="s"),
    out_type=jax.ShapeDtypeStruct((_N_WORKERS, _N_BINS), jnp.float32),
    scratch_types=[
        pltpu.VMEM((_IDS_PER_WORKER,), jnp.int32),
        pltpu.VMEM((_N_BINS,), jnp.float32),
    ],
    compiler_params=pltpu.CompilerParams(needs_layout_passes=False,
                                         disable_bounds_checks=True),
)


def _mlp_body(pc_ref, emb_ref, w1_ref, b1_ref, w2_ref, b2_ref, out_ref):
    counts = jnp.sum(pc_ref[...], axis=0, keepdims=True)
    e = lax.dot_general(counts, emb_ref[...],
                        (((1,), (0,)), ((), ())),
                        preferred_element_type=jnp.float32) * (1.0 / _N_IDS)
    h = lax.dot_general(e, w1_ref[...],
                        (((1,), (1,)), ((), ())),
                        preferred_element_type=jnp.float32) + b1_ref[...]
    h = jnp.maximum(h, 0.0)
    v = lax.dot_general(h, w2_ref[...],
                        (((1,), (1,)), ((), ())),
                        preferred_element_type=jnp.float32) + b2_ref[...]
    out_ref[...] = v


def kernel(ids, emb, W1, b1, W2, b2):
    ids32 = ids.astype(jnp.int32)
    partial_counts = _hist(ids32)
    out = pl.pallas_call(
        _mlp_body,
        out_shape=jax.ShapeDtypeStruct((1, 8), jnp.float32),
    )(partial_counts, emb, W1, b1.reshape(1, 64), W2, b2.reshape(1, 8))
    return out[0]
